# baseline (speedup 1.0000x reference)
import jax
import jax.numpy as jnp
from jax import lax
from jax.experimental import pallas as pl
from jax.experimental.pallas import tpu as pltpu
from jax.experimental.pallas import tpu_sc as plsc

N = 50000
D = 64
E = 800000
FQ = 16
NQ = 4
NC = 2
NS = 16
L = 16

NP = 51200
RT = NP // NS
RB = 128
NRB = RT // RB
EB = 1536
ETB = 34
EP = NS * EB * ETB
EPT = EP // NS


def _rsqrt16(d):
    xi = lax.bitcast_convert_type(d, jnp.int32)
    xi = jnp.int32(0x5F3759DF) - lax.shift_right_arithmetic(xi, 1)
    r = lax.bitcast_convert_type(xi, jnp.float32)
    for _ in range(3):
        r = r * (1.5 - 0.5 * d * r * r)
    return jnp.where(d > 0, r, 0.0)


def _body(xs, srcb, dstb, out, y0, h1, yy,
          acc, degs, sidx, didx, rows, dis_t, blk_a, blk_b, blk_c,
          zrow, ones_v, zvec, gsem, ssem):
    c = lax.axis_index("c")
    t = lax.axis_index("s")
    z16 = jnp.zeros((L,), jnp.float32)

    def _init_row(i, _):
        zrow[i, pl.ds(0, L)] = z16
        return 0
    lax.fori_loop(0, RB, _init_row, 0)

    def _init_v(i, _):
        zvec[pl.ds(i * L, L)] = z16
        return 0
    lax.fori_loop(0, 128 // L, _init_v, 0)

    def _init_ones(i, _):
        ones_v[pl.ds(i * L, L)] = jnp.full((L,), 1.0, jnp.float32)
        return 0
    lax.fori_loop(0, EB // L, _init_ones, 0)

    scope = jax.named_scope
    def _zero_acc(b, _):
        pltpu.sync_copy(zrow, acc.at[pl.ds(t * RT + b * RB, RB)])
        pltpu.sync_copy(zvec, degs.at[pl.ds(t * RT + b * RB, RB)])
        return 0
    with scope("ph_zero"):
        lax.fori_loop(0, NRB, _zero_acc, 0)
        plsc.subcore_barrier()

    def _deg_blk(bb, _):
        for u in (0, 1):
            b = bb * 2 + u
            pn = 1 - u

            @pl.when(b >= 1)
            def _w():
                pltpu.make_async_copy(ones_v, degs.at[didx.at[pn]],
                                      ssem).wait()
            pltpu.async_copy(ones_v, degs.at[didx.at[u]], ssem, add=True)

            @pl.when(b + 1 < ETB)
            def _p():
                pltpu.sync_copy(
                    dstb.at[pl.ds(t * EPT + (b + 1) * EB, EB)], didx.at[pn])
        return 0
    with scope("ph_deg"):
        pltpu.sync_copy(dstb.at[pl.ds(t * EPT, EB)], didx.at[0])
        lax.fori_loop(0, ETB // 2, _deg_blk, 0)
        pltpu.make_async_copy(ones_v, degs.at[didx.at[(ETB - 1) % 2]],
                              ssem).wait()
        plsc.subcore_barrier()

    def _dis_blk(i, _):
        d = dis_t[pl.ds(i * L, L)]
        dis_t[pl.ds(i * L, L)] = _rsqrt16(d)
        return 0
    with scope("ph_dis"):
        pltpu.sync_copy(degs.at[pl.ds(t * RT, RT)], dis_t)
        lax.fori_loop(0, RT // L, _dis_blk, 0)

    def _spass(ysrc):
        ebase0 = t * EPT
        pltpu.sync_copy(srcb.at[pl.ds(ebase0, EB)], sidx.at[0])
        pltpu.sync_copy(dstb.at[pl.ds(ebase0, EB)], didx.at[0])
        pltpu.async_copy(ysrc.at[sidx.at[0]], rows.at[0], gsem)

        def _edge_blk(bb, _):
            for u in (0, 1):
                b = bb * 2 + u
                pn = 1 - u

                @pl.when(b >= 1)
                def _w():
                    pltpu.make_async_copy(rows.at[pn], acc.at[didx.at[pn]],
                                          ssem).wait()

                @pl.when(b + 1 < ETB)
                def _p():
                    eb1 = ebase0 + (b + 1) * EB
                    pltpu.sync_copy(srcb.at[pl.ds(eb1, EB)], sidx.at[pn])
                    pltpu.sync_copy(dstb.at[pl.ds(eb1, EB)], didx.at[pn])
                    pltpu.async_copy(ysrc.at[sidx.at[pn]], rows.at[pn], gsem)

                pltpu.make_async_copy(ysrc.at[sidx.at[u]], rows.at[u],
                                      gsem).wait()
                pltpu.async_copy(rows.at[u], acc.at[didx.at[u]], ssem,
                                 add=True)
            return 0
        with scope("ph_spass"):
            lax.fori_loop(0, ETB // 2, _edge_blk, 0)
            lastp = (ETB - 1) % 2
            pltpu.make_async_copy(rows.at[lastp], acc.at[didx.at[lastp]],
                                  ssem).wait()
            plsc.subcore_barrier()

    def _dv(il):
        return plsc.load_gather(dis_t, [jnp.full((L,), il, jnp.int32)])

    def _quarter(p, _):
        q = c * 2 + p
        xq = xs.at[q]
        y0q = y0.at[q]
        h1q = h1.at[q]
        yyq = yy.at[q]
        outq = out.at[q]

        def _scale_blk(b, _):
            gbase = t * RT + b * RB
            pltpu.sync_copy(xq.at[pl.ds(gbase, RB)], blk_a)

            def _row(r, _):
                dv = _dv(b * RB + r)
                blk_b[r, pl.ds(0, L)] = blk_a[r, pl.ds(0, L)] * dv
                return 0
            lax.fori_loop(0, RB, _row, 0)
            pltpu.sync_copy(blk_b, y0q.at[pl.ds(gbase, RB)])
            return 0
        with scope("ph_scale"):
            lax.fori_loop(0, NRB, _scale_blk, 0)
            plsc.subcore_barrier()

        _spass(y0q)

        def _ep1_blk(b, _):
            gbase = t * RT + b * RB
            pltpu.sync_copy(acc.at[pl.ds(gbase, RB)], blk_a)
            pltpu.sync_copy(zrow, acc.at[pl.ds(gbase, RB)])

            def _row(r, _):
                dv = _dv(b * RB + r)
                hv = blk_a[r, pl.ds(0, L)] * dv
                blk_b[r, pl.ds(0, L)] = hv
                blk_c[r, pl.ds(0, L)] = hv * dv
                return 0
            lax.fori_loop(0, RB, _row, 0)
            pltpu.sync_copy(blk_b, h1q.at[pl.ds(gbase, RB)])
            pltpu.sync_copy(blk_c, yyq.at[pl.ds(gbase, RB)])
            return 0
        with scope("ph_ep1"):
            lax.fori_loop(0, NRB, _ep1_blk, 0)
            plsc.subcore_barrier()

        _spass(yyq)

        def _ep2_blk(b, _):
            gbase = t * RT + b * RB
            pltpu.sync_copy(acc.at[pl.ds(gbase, RB)], blk_a)
            pltpu.sync_copy(zrow, acc.at[pl.ds(gbase, RB)])
            pltpu.sync_copy(xq.at[pl.ds(gbase, RB)], blk_b)
            pltpu.sync_copy(h1q.at[pl.ds(gbase, RB)], blk_c)

            def _row(r, _):
                dv = _dv(b * RB + r)
                third = jnp.float32(1.0 / 3.0)
                z2 = blk_a[r, pl.ds(0, L)]
                xv = blk_b[r, pl.ds(0, L)]
                hv = blk_c[r, pl.ds(0, L)]
                blk_a[r, pl.ds(0, L)] = (xv + hv + z2 * dv) * third
                return 0
            lax.fori_loop(0, RB, _row, 0)
            pltpu.sync_copy(blk_a, outq.at[pl.ds(gbase, RB)])
            return 0
        with scope("ph_ep2"):
            lax.fori_loop(0, NRB, _ep2_blk, 0)
            plsc.subcore_barrier()
        return 0

    lax.fori_loop(0, 2, _quarter, 0)


_mesh = plsc.VectorSubcoreMesh(
    core_axis_name="c", subcore_axis_name="s", num_cores=NC, num_subcores=NS)

_qbuf = jax.ShapeDtypeStruct((NQ, NP, FQ), jnp.float32)

_gcn = pl.kernel(
    _body,
    out_type=(_qbuf, _qbuf, _qbuf, _qbuf),
    mesh=_mesh,
    scratch_types=[
        pltpu.VMEM_SHARED((NP, FQ), jnp.float32),
        pltpu.VMEM_SHARED((NP,), jnp.float32),
        pltpu.VMEM((2, EB), jnp.int32),
        pltpu.VMEM((2, EB), jnp.int32),
        pltpu.VMEM((2, EB, FQ), jnp.float32),
        pltpu.VMEM((RT,), jnp.float32),
        pltpu.VMEM((RB, FQ), jnp.float32),
        pltpu.VMEM((RB, FQ), jnp.float32),
        pltpu.VMEM((RB, FQ), jnp.float32),
        pltpu.VMEM((RB, FQ), jnp.float32),
        pltpu.VMEM((EB,), jnp.float32),
        pltpu.VMEM((128,), jnp.float32),
        pltpu.SemaphoreType.DMA,
        pltpu.SemaphoreType.DMA,
    ],
    compiler_params=pltpu.CompilerParams(
        needs_layout_passes=False, use_tc_tiling_on_sc=False),
)


@jax.jit
def kernel(x, edge_index):
    src = edge_index[0].astype(jnp.int32)
    dst = edge_index[1].astype(jnp.int32)
    pad = jnp.full((EP - E,), N, jnp.int32)
    srcb = jnp.concatenate([src, pad])
    dstb = jnp.concatenate([dst, pad])
    xp = jnp.pad(x, ((0, NP - N), (0, 0)))
    xs = jnp.transpose(xp.reshape(NP, NQ, FQ), (1, 0, 2))
    out, _, _, _ = _gcn(xs, srcb, dstb)
    return out[:, :N, :].transpose(1, 0, 2).reshape(N, D)


# speedup vs baseline: 1.4219x; 1.4219x over previous
import jax
import jax.numpy as jnp
from jax import lax
from jax.experimental import pallas as pl
from jax.experimental.pallas import tpu as pltpu
from jax.experimental.pallas import tpu_sc as plsc

N = 50000
D = 64
E = 800000
F = 32
NC = 2
NS = 16
L = 16

NP = 50176
RT = NP // NS
RB = 64
NRB = RT // RB
EB = 576
ETB = 88
EPT = EB * ETB
EP = NS * EPT


def _rsqrt16(d):
    xi = lax.bitcast_convert_type(d, jnp.int32)
    xi = jnp.int32(0x5F3759DF) - lax.shift_right_arithmetic(xi, 1)
    r = lax.bitcast_convert_type(xi, jnp.float32)
    for _ in range(3):
        r = r * (1.5 - 0.5 * d * r * r)
    return jnp.where(d > 0, r, 0.0)


def _body(xs, srcb, dstb, out, y0, h1, yy,
          acc, degs, sidx, didx, rows, dis_t, zrow, gsem, ssem):
    c = lax.axis_index("c")
    t = lax.axis_index("s")
    z16 = jnp.zeros((L,), jnp.float32)
    scope = jax.named_scope

    xc = xs.at[c]
    y0c = y0.at[c]
    h1c = h1.at[c]
    yyc = yy.at[c]
    outc = out.at[c]

    blk_a = rows.at[pl.ds(0, RB)]
    blk_b = rows.at[pl.ds(RB, RB)]
    blk_c = rows.at[pl.ds(2 * RB, RB)]

    def _init_zrow(i, _):
        zrow[i, pl.ds(0, L)] = z16
        zrow[i, pl.ds(L, L)] = z16
        return 0
    lax.fori_loop(0, RB, _init_zrow, 0)

    def _init_dis(i, _):
        dis_t[pl.ds(i * L, L)] = z16
        return 0
    lax.fori_loop(0, RT // L, _init_dis, 0)

    def _zero_acc(b, _):
        pltpu.sync_copy(zrow, acc.at[pl.ds(t * RT + b * RB, RB)])
        return 0

    with scope("ph_zero"):
        lax.fori_loop(0, NRB, _zero_acc, 0)
        pltpu.sync_copy(dis_t, degs.at[pl.ds(t * RT, RT)])
        plsc.subcore_barrier()

    def _init_ones(i, _):
        dis_t[pl.ds(i * L, L)] = jnp.full((L,), 1.0, jnp.float32)
        return 0
    lax.fori_loop(0, EB // L, _init_ones, 0)
    ones_v = dis_t.at[pl.ds(0, EB)]

    def _deg_blk(b, _):
        pltpu.sync_copy(dstb.at[pl.ds(t * EPT + b * EB, EB)], didx)
        pltpu.sync_copy(ones_v, degs.at[didx], add=True)
        return 0

    with scope("ph_deg"):
        lax.fori_loop(0, ETB, _deg_blk, 0)
        plsc.subcore_barrier()

    def _dis_blk(i, _):
        d = dis_t[pl.ds(i * L, L)]
        dis_t[pl.ds(i * L, L)] = _rsqrt16(d)
        return 0

    with scope("ph_dis"):
        pltpu.sync_copy(degs.at[pl.ds(t * RT, RT)], dis_t)
        lax.fori_loop(0, RT // L, _dis_blk, 0)

    def _dv(il):
        return plsc.load_gather(dis_t, [jnp.full((L,), il, jnp.int32)])

    def _scale_blk(b, _):
        gbase = t * RT + b * RB
        pltpu.sync_copy(xc.at[pl.ds(gbase, RB)], blk_a)

        def _row(r, _):
            dv = _dv(b * RB + r)
            blk_b[r, pl.ds(0, L)] = blk_a[r, pl.ds(0, L)] * dv
            blk_b[r, pl.ds(L, L)] = blk_a[r, pl.ds(L, L)] * dv
            return 0
        lax.fori_loop(0, RB, _row, 0)
        pltpu.sync_copy(blk_b, y0c.at[pl.ds(gbase, RB)])
        return 0

    with scope("ph_scale"):
        lax.fori_loop(0, NRB, _scale_blk, 0)
        plsc.subcore_barrier()

    def _spass(ysrc):
        def _edge_blk(b, _):
            ebase = t * EPT + b * EB
            pltpu.sync_copy(srcb.at[pl.ds(ebase, EB)], sidx)
            pltpu.sync_copy(dstb.at[pl.ds(ebase, EB)], didx)
            pltpu.async_copy(ysrc.at[sidx], rows, gsem).wait()
            pltpu.async_copy(rows, acc.at[didx], ssem, add=True).wait()
            return 0

        with scope("ph_spass"):
            lax.fori_loop(0, ETB, _edge_blk, 0)
            plsc.subcore_barrier()

    _spass(y0c)

    def _ep1_blk(b, _):
        gbase = t * RT + b * RB
        pltpu.sync_copy(acc.at[pl.ds(gbase, RB)], blk_a)
        pltpu.sync_copy(zrow, acc.at[pl.ds(gbase, RB)])

        def _row(r, _):
            dv = _dv(b * RB + r)
            for h in range(2):
                hv = blk_a[r, pl.ds(h * L, L)] * dv
                blk_b[r, pl.ds(h * L, L)] = hv
                blk_c[r, pl.ds(h * L, L)] = hv * dv
            return 0
        lax.fori_loop(0, RB, _row, 0)
        pltpu.sync_copy(blk_b, h1c.at[pl.ds(gbase, RB)])
        pltpu.sync_copy(blk_c, yyc.at[pl.ds(gbase, RB)])
        return 0

    with scope("ph_ep1"):
        lax.fori_loop(0, NRB, _ep1_blk, 0)
        plsc.subcore_barrier()

    _spass(yyc)

    def _ep2_blk(b, _):
        gbase = t * RT + b * RB
        pltpu.sync_copy(acc.at[pl.ds(gbase, RB)], blk_a)
        pltpu.sync_copy(xc.at[pl.ds(gbase, RB)], blk_b)
        pltpu.sync_copy(h1c.at[pl.ds(gbase, RB)], blk_c)

        def _row(r, _):
            dv = _dv(b * RB + r)
            third = jnp.float32(1.0 / 3.0)
            for h in range(2):
                z2 = blk_a[r, pl.ds(h * L, L)]
                xv = blk_b[r, pl.ds(h * L, L)]
                hv = blk_c[r, pl.ds(h * L, L)]
                blk_a[r, pl.ds(h * L, L)] = (xv + hv + z2 * dv) * third
            return 0
        lax.fori_loop(0, RB, _row, 0)
        pltpu.sync_copy(blk_a, outc.at[pl.ds(gbase, RB)])
        return 0

    with scope("ph_ep2"):
        lax.fori_loop(0, NRB, _ep2_blk, 0)


_mesh = plsc.VectorSubcoreMesh(
    core_axis_name="c", subcore_axis_name="s", num_cores=NC, num_subcores=NS)

_half = jax.ShapeDtypeStruct((NC, NP, F), jnp.float32)

_gcn = pl.kernel(
    _body,
    out_type=(_half, _half, _half, _half),
    mesh=_mesh,
    scratch_types=[
        pltpu.VMEM_SHARED((NP, F), jnp.float32),
        pltpu.VMEM_SHARED((NP,), jnp.float32),
        pltpu.VMEM((EB,), jnp.int32),
        pltpu.VMEM((EB,), jnp.int32),
        pltpu.VMEM((EB, F), jnp.float32),
        pltpu.VMEM((RT,), jnp.float32),
        pltpu.VMEM((RB, F), jnp.float32),
        pltpu.SemaphoreType.DMA,
        pltpu.SemaphoreType.DMA,
    ],
    compiler_params=pltpu.CompilerParams(
        needs_layout_passes=False, use_tc_tiling_on_sc=False),
)


@jax.jit
def kernel(x, edge_index):
    src = edge_index[0].astype(jnp.int32)
    dst = edge_index[1].astype(jnp.int32)
    pad = jnp.full((EP - E,), N, jnp.int32)
    srcb = jnp.concatenate([src, pad])
    dstb = jnp.concatenate([dst, pad])
    xp = jnp.pad(x, ((0, NP - N), (0, 0)))
    xs = jnp.stack([xp[:, :F], xp[:, F:]])
    out, _, _, _ = _gcn(xs, srcb, dstb)
    return jnp.concatenate([out[0, :N], out[1, :N]], axis=1)


# speedup vs baseline: 1.5963x; 1.1226x over previous
import jax
import jax.numpy as jnp
from jax import lax
from jax.experimental import pallas as pl
from jax.experimental.pallas import tpu as pltpu
from jax.experimental.pallas import tpu_sc as plsc

N = 50000
D = 64
E = 800000
F = 32
NC = 2
NS = 16
L = 16

NP = 51200
RT = NP // NS
RB = 128
NRB = RT // RB
EPT = E // NS
EB = 400
NEB = EPT // EB


def _rsqrt16(d):
    xi = lax.bitcast_convert_type(d, jnp.int32)
    xi = jnp.int32(0x5F3759DF) - lax.shift_right_arithmetic(xi, 1)
    r = lax.bitcast_convert_type(xi, jnp.float32)
    for _ in range(3):
        r = r * (1.5 - 0.5 * d * r * r)
    return jnp.where(d > 0, r, 0.0)


def _body(xs, srcb, dstb, out, y0, h1, yy,
          acc, degs, sidx, didx, rows, dis_t, zrow, gsem, ssem):
    c = lax.axis_index("c")
    t = lax.axis_index("s")
    z16 = jnp.zeros((L,), jnp.float32)
    scope = jax.named_scope

    xc = xs.at[c]
    y0c = y0.at[c]
    h1c = h1.at[c]
    yyc = yy.at[c]
    outc = out.at[c]
    nbase = t * RT
    ebase = t * EPT

    blk_a = rows.at[pl.ds(0, RB)]
    blk_b = rows.at[pl.ds(RB, RB)]
    blk_c = rows.at[pl.ds(2 * RB, RB)]

    def _dv(il):
        return plsc.load_gather(dis_t, [jnp.full((L,), il, jnp.int32)])

    def _init_zrow(i, _):
        zrow[i, pl.ds(0, L)] = z16
        zrow[i, pl.ds(L, L)] = z16
        return 0
    lax.fori_loop(0, RB, _init_zrow, 0)

    def _init_dis(i, _):
        dis_t[pl.ds(i * L, L)] = z16
        return 0
    lax.fori_loop(0, RT // L, _init_dis, 0)

    def _zero_acc(b, _):
        pltpu.sync_copy(zrow, acc.at[pl.ds(nbase + b * RB, RB)])
        return 0

    with scope("ph_zero"):
        lax.fori_loop(0, NRB, _zero_acc, 0)
        pltpu.sync_copy(dis_t, degs.at[pl.ds(nbase, RT)])
        plsc.subcore_barrier()

    def _init_ones(i, _):
        dis_t[pl.ds(i * L, L)] = jnp.full((L,), 1.0, jnp.float32)
        return 0
    lax.fori_loop(0, EB // L, _init_ones, 0)

    def _deg_blk(b, _):
        pltpu.sync_copy(dstb.at[pl.ds(ebase + b * EB, EB)], didx)
        pltpu.sync_copy(dis_t.at[pl.ds(0, EB)], degs.at[didx], add=True)
        return 0

    with scope("ph_deg"):
        lax.fori_loop(0, NEB, _deg_blk, 0)
        plsc.subcore_barrier()

    def _dis_blk(i, _):
        d = dis_t[pl.ds(i * L, L)]
        dis_t[pl.ds(i * L, L)] = _rsqrt16(d)
        return 0

    with scope("ph_dis"):
        pltpu.sync_copy(degs.at[pl.ds(nbase, RT)], dis_t)
        lax.fori_loop(0, RT // L, _dis_blk, 0)

    def _scale_blk(b, _):
        gbase = nbase + b * RB
        pltpu.sync_copy(xc.at[pl.ds(gbase, RB)], blk_a)

        def _row(r, _):
            dv = _dv(b * RB + r)
            blk_b[r, pl.ds(0, L)] = blk_a[r, pl.ds(0, L)] * dv
            blk_b[r, pl.ds(L, L)] = blk_a[r, pl.ds(L, L)] * dv
            return 0
        lax.fori_loop(0, RB, _row, 0)
        pltpu.sync_copy(blk_b, y0c.at[pl.ds(gbase, RB)])
        return 0

    with scope("ph_scale"):
        lax.fori_loop(0, NRB, _scale_blk, 0)
        plsc.subcore_barrier()

    def _spass(ysrc):
        def _edge_blk(b, _):
            eb0 = ebase + b * EB
            pltpu.sync_copy(srcb.at[pl.ds(eb0, EB)], sidx)
            pltpu.sync_copy(dstb.at[pl.ds(eb0, EB)], didx)
            pltpu.async_copy(ysrc.at[sidx], rows, gsem).wait()
            pltpu.async_copy(rows, acc.at[didx], ssem, add=True).wait()
            return 0

        with scope("ph_spass"):
            lax.fori_loop(0, NEB, _edge_blk, 0)
            plsc.subcore_barrier()

    _spass(y0c)

    def _ep1_blk(b, _):
        gbase = nbase + b * RB
        pltpu.sync_copy(acc.at[pl.ds(gbase, RB)], blk_a)
        pltpu.sync_copy(zrow, acc.at[pl.ds(gbase, RB)])

        def _row(r, _):
            dv = _dv(b * RB + r)
            for h in range(2):
                hv = blk_a[r, pl.ds(h * L, L)] * dv
                blk_b[r, pl.ds(h * L, L)] = hv
                blk_c[r, pl.ds(h * L, L)] = hv * dv
            return 0
        lax.fori_loop(0, RB, _row, 0)
        pltpu.sync_copy(blk_b, h1c.at[pl.ds(gbase, RB)])
        pltpu.sync_copy(blk_c, yyc.at[pl.ds(gbase, RB)])
        return 0

    with scope("ph_ep1"):
        lax.fori_loop(0, NRB, _ep1_blk, 0)
        plsc.subcore_barrier()

    _spass(yyc)

    def _ep2_blk(b, _):
        gbase = nbase + b * RB
        pltpu.sync_copy(acc.at[pl.ds(gbase, RB)], blk_a)
        pltpu.sync_copy(xc.at[pl.ds(gbase, RB)], blk_b)
        pltpu.sync_copy(h1c.at[pl.ds(gbase, RB)], blk_c)

        def _row(r, _):
            dv = _dv(b * RB + r)
            third = jnp.float32(1.0 / 3.0)
            for h in range(2):
                z2 = blk_a[r, pl.ds(h * L, L)]
                xv = blk_b[r, pl.ds(h * L, L)]
                hv = blk_c[r, pl.ds(h * L, L)]
                blk_a[r, pl.ds(h * L, L)] = (xv + hv + z2 * dv) * third
            return 0
        lax.fori_loop(0, RB, _row, 0)
        pltpu.sync_copy(blk_a, outc.at[pl.ds(gbase, RB)])
        return 0

    with scope("ph_ep2"):
        lax.fori_loop(0, NRB, _ep2_blk, 0)


_mesh = plsc.VectorSubcoreMesh(
    core_axis_name="c", subcore_axis_name="s", num_cores=NC, num_subcores=NS)

_half = jax.ShapeDtypeStruct((NC, NP, F), jnp.float32)

_gcn = pl.kernel(
    _body,
    out_type=(_half, _half, _half, _half),
    mesh=_mesh,
    scratch_types=[
        pltpu.VMEM_SHARED((NP, F), jnp.float32),
        pltpu.VMEM_SHARED((NP,), jnp.float32),
        pltpu.VMEM((EB,), jnp.int32),
        pltpu.VMEM((EB,), jnp.int32),
        pltpu.VMEM((EB, F), jnp.float32),
        pltpu.VMEM((RT,), jnp.float32),
        pltpu.VMEM((RB, F), jnp.float32),
        pltpu.SemaphoreType.DMA,
        pltpu.SemaphoreType.DMA,
    ],
    compiler_params=pltpu.CompilerParams(
        needs_layout_passes=False, use_tc_tiling_on_sc=False),
)


@jax.jit
def kernel(x, edge_index):
    src = edge_index[0].astype(jnp.int32)
    dst = edge_index[1].astype(jnp.int32)
    xp = jnp.pad(x, ((0, NP - N), (0, 0)))
    xs = jnp.stack([xp[:, :F], xp[:, F:]])
    out, _, _, _ = _gcn(xs, src, dst)
    return jnp.concatenate([out[0, :N], out[1, :N]], axis=1)


# speedup vs baseline: 1.6727x; 1.0479x over previous
import jax
import jax.numpy as jnp
from jax import lax
from jax.experimental import pallas as pl
from jax.experimental.pallas import tpu as pltpu
from jax.experimental.pallas import tpu_sc as plsc

N = 50000
D = 64
E = 800000
F = 32
NC = 2
NS = 16
L = 16

NP = 51200
RT = NP // NS
RB = 128
NRB = RT // RB
EPT = E // NS
EB = 400
NEB = EPT // EB
EBD = 2000
NEBD = EPT // EBD


def _rsqrt16(d):
    xi = lax.bitcast_convert_type(d, jnp.int32)
    xi = jnp.int32(0x5F3759DF) - lax.shift_right_arithmetic(xi, 1)
    r = lax.bitcast_convert_type(xi, jnp.float32)
    for _ in range(3):
        r = r * (1.5 - 0.5 * d * r * r)
    return jnp.where(d > 0, r, 0.0)


def _body(xs, srcb, dstb, out, y0, h1, yy,
          acc, degs, sidx, didx, didx_d, rows, dis_t, zrow, gsem, ssem):
    c = lax.axis_index("c")
    t = lax.axis_index("s")
    z16 = jnp.zeros((L,), jnp.float32)
    scope = jax.named_scope

    xc = xs.at[c]
    y0c = y0.at[c]
    h1c = h1.at[c]
    yyc = yy.at[c]
    outc = out.at[c]
    nbase = t * RT
    ebase = t * EPT

    blk_a = rows.at[pl.ds(0, RB)]
    blk_b = rows.at[pl.ds(RB, RB)]
    blk_c = rows.at[pl.ds(2 * RB, RB)]

    def _dv(il):
        return plsc.load_gather(dis_t, [jnp.full((L,), il, jnp.int32)])

    def _init_zrow(i, _):
        zrow[i, pl.ds(0, L)] = z16
        zrow[i, pl.ds(L, L)] = z16
        return 0
    lax.fori_loop(0, RB, _init_zrow, 0)

    def _init_dis(i, _):
        dis_t[pl.ds(i * L, L)] = z16
        return 0
    lax.fori_loop(0, RT // L, _init_dis, 0)

    def _zero_acc(b, _):
        pltpu.sync_copy(zrow, acc.at[pl.ds(nbase + b * RB, RB)])
        return 0

    with scope("ph_zero"):
        lax.fori_loop(0, NRB, _zero_acc, 0)
        pltpu.sync_copy(dis_t, degs.at[pl.ds(nbase, RT)])
        plsc.subcore_barrier()

    def _init_ones(i, _):
        dis_t[pl.ds(i * L, L)] = jnp.full((L,), 1.0, jnp.float32)
        return 0
    lax.fori_loop(0, EBD // L, _init_ones, 0)

    def _deg_blk(b, _):
        pltpu.sync_copy(dstb.at[pl.ds(ebase + b * EBD, EBD)], didx_d)
        pltpu.sync_copy(dis_t.at[pl.ds(0, EBD)], degs.at[didx_d], add=True)
        return 0

    with scope("ph_deg"):
        lax.fori_loop(0, NEBD, _deg_blk, 0)
        plsc.subcore_barrier()

    def _dis_blk(i, _):
        d = dis_t[pl.ds(i * L, L)]
        dis_t[pl.ds(i * L, L)] = _rsqrt16(d)
        return 0

    with scope("ph_dis"):
        pltpu.sync_copy(degs.at[pl.ds(nbase, RT)], dis_t)
        lax.fori_loop(0, RT // L, _dis_blk, 0)

    def _scale_blk(b, _):
        gbase = nbase + b * RB
        pltpu.sync_copy(xc.at[pl.ds(gbase, RB)], blk_a)

        def _row(r, _):
            dv = _dv(b * RB + r)
            blk_b[r, pl.ds(0, L)] = blk_a[r, pl.ds(0, L)] * dv
            blk_b[r, pl.ds(L, L)] = blk_a[r, pl.ds(L, L)] * dv
            return 0
        lax.fori_loop(0, RB, _row, 0)
        pltpu.sync_copy(blk_b, y0c.at[pl.ds(gbase, RB)])
        return 0

    with scope("ph_scale"):
        lax.fori_loop(0, NRB, _scale_blk, 0)
        plsc.subcore_barrier()

    def _spass(ysrc):
        def _edge_blk(b, _):
            eb0 = ebase + b * EB
            pltpu.sync_copy(srcb.at[pl.ds(eb0, EB)], sidx)
            pltpu.sync_copy(dstb.at[pl.ds(eb0, EB)], didx)
            pltpu.async_copy(ysrc.at[sidx], rows, gsem).wait()
            pltpu.async_copy(rows, acc.at[didx], ssem, add=True).wait()
            return 0

        with scope("ph_spass"):
            lax.fori_loop(0, NEB, _edge_blk, 0)
            plsc.subcore_barrier()

    _spass(y0c)

    def _ep1_blk(b, _):
        gbase = nbase + b * RB
        pltpu.sync_copy(acc.at[pl.ds(gbase, RB)], blk_a)
        pltpu.sync_copy(zrow, acc.at[pl.ds(gbase, RB)])

        def _row(r, _):
            dv = _dv(b * RB + r)
            for h in range(2):
                hv = blk_a[r, pl.ds(h * L, L)] * dv
                blk_b[r, pl.ds(h * L, L)] = hv
                blk_c[r, pl.ds(h * L, L)] = hv * dv
            return 0
        lax.fori_loop(0, RB, _row, 0)
        pltpu.sync_copy(blk_b, h1c.at[pl.ds(gbase, RB)])
        pltpu.sync_copy(blk_c, yyc.at[pl.ds(gbase, RB)])
        return 0

    with scope("ph_ep1"):
        lax.fori_loop(0, NRB, _ep1_blk, 0)
        plsc.subcore_barrier()

    _spass(yyc)

    def _ep2_blk(b, _):
        gbase = nbase + b * RB
        pltpu.sync_copy(acc.at[pl.ds(gbase, RB)], blk_a)
        pltpu.sync_copy(xc.at[pl.ds(gbase, RB)], blk_b)
        pltpu.sync_copy(h1c.at[pl.ds(gbase, RB)], blk_c)

        def _row(r, _):
            dv = _dv(b * RB + r)
            third = jnp.float32(1.0 / 3.0)
            for h in range(2):
                z2 = blk_a[r, pl.ds(h * L, L)]
                xv = blk_b[r, pl.ds(h * L, L)]
                hv = blk_c[r, pl.ds(h * L, L)]
                blk_a[r, pl.ds(h * L, L)] = (xv + hv + z2 * dv) * third
            return 0
        lax.fori_loop(0, RB, _row, 0)
        pltpu.sync_copy(blk_a, outc.at[pl.ds(gbase, RB)])
        return 0

    with scope("ph_ep2"):
        lax.fori_loop(0, NRB, _ep2_blk, 0)


_mesh = plsc.VectorSubcoreMesh(
    core_axis_name="c", subcore_axis_name="s", num_cores=NC, num_subcores=NS)

_half = jax.ShapeDtypeStruct((NC, NP, F), jnp.float32)

_gcn = pl.kernel(
    _body,
    out_type=(_half, _half, _half, _half),
    mesh=_mesh,
    scratch_types=[
        pltpu.VMEM_SHARED((NP, F), jnp.float32),
        pltpu.VMEM_SHARED((NP,), jnp.float32),
        pltpu.VMEM((EB,), jnp.int32),
        pltpu.VMEM((EB,), jnp.int32),
        pltpu.VMEM((EBD,), jnp.int32),
        pltpu.VMEM((EB, F), jnp.float32),
        pltpu.VMEM((RT,), jnp.float32),
        pltpu.VMEM((RB, F), jnp.float32),
        pltpu.SemaphoreType.DMA,
        pltpu.SemaphoreType.DMA,
    ],
    compiler_params=pltpu.CompilerParams(
        needs_layout_passes=False, use_tc_tiling_on_sc=False),
)


@jax.jit
def kernel(x, edge_index):
    src = edge_index[0].astype(jnp.int32)
    dst = edge_index[1].astype(jnp.int32)
    xp = jnp.pad(x, ((0, NP - N), (0, 0)))
    xs = jnp.stack([xp[:, :F], xp[:, F:]])
    out, _, _, _ = _gcn(xs, src, dst)
    return jnp.concatenate([out[0, :N], out[1, :N]], axis=1)
